# Initial kernel scaffold; baseline (speedup 1.0000x reference)
import jax
import jax.numpy as jnp
from jax.experimental import pallas as pl


def kernel(x, edge_index, edge_weight, batch, W0, b0, W1, b1, g0, be0, g1, be1, Wp0, bp0, Wp1, bp1, Wp2, bp2):
    raise NotImplementedError("write your pallas kernel here")


# speedup vs baseline: 2.8712x; 2.8712x over previous
import functools

import jax
import jax.numpy as jnp
from jax import lax
from jax.experimental import pallas as pl
from jax.experimental.pallas import tpu as pltpu
from jax.experimental.pallas import tpu_sc as plsc

N = 10000
E = 160000
D = 256
DH = 128
D_OUT = 128
G = 64

NB = 10
RB = N // NB

NTILES = 16
EPT = E // NTILES
CHUNK = 80
NCHUNK = EPT // CHUNK
NPAD = 10240
RPT = NPAD // NTILES
ZROWS = 128


def _mm0_body(x_ref, w_ref, o_ref):
    res = jnp.dot(x_ref[...], w_ref[...], preferred_element_type=jnp.float32)
    o_ref[0, :, :] = res[:, :DH]
    o_ref[1, :, :] = res[:, DH:]


def _mm0(x, w):
    return pl.pallas_call(
        _mm0_body,
        grid=(NB,),
        in_specs=[
            pl.BlockSpec((RB, D), lambda i: (i, 0)),
            pl.BlockSpec((D, D), lambda i: (0, 0)),
        ],
        out_specs=pl.BlockSpec((2, RB, DH), lambda i: (0, i, 0)),
        out_shape=jax.ShapeDtypeStruct((2, N, DH), jnp.float32),
    )(x, w)


def _mm1_body(c_ref, s_ref, t_ref, w_ref, o_ref):
    h_lo = jnp.maximum(c_ref[0] * s_ref[0] + t_ref[0], 0.0)
    h_hi = jnp.maximum(c_ref[1] * s_ref[1] + t_ref[1], 0.0)
    h = jnp.concatenate([h_lo, h_hi], axis=1)
    res = jnp.dot(h, w_ref[...], preferred_element_type=jnp.float32)
    o_ref[0, :, :] = res[:, :DH]
    o_ref[1, :, :] = res[:, DH:]


def _mm1(conv, scale, shift, w):
    return pl.pallas_call(
        _mm1_body,
        grid=(NB,),
        in_specs=[
            pl.BlockSpec((2, RB, DH), lambda i: (0, i, 0)),
            pl.BlockSpec((2, 1, DH), lambda i: (0, 0, 0)),
            pl.BlockSpec((2, 1, DH), lambda i: (0, 0, 0)),
            pl.BlockSpec((D, D), lambda i: (0, 0)),
        ],
        out_specs=pl.BlockSpec((2, RB, DH), lambda i: (0, i, 0)),
        out_shape=jax.ShapeDtypeStruct((2, N, DH), jnp.float32),
    )(conv, scale, shift, w)


def _stats_body(c_ref, g_ref, be_ref, s_ref, t_ref, acc1, acc2):
    i = pl.program_id(0)

    @pl.when(i == 0)
    def _():
        acc1[...] = jnp.zeros_like(acc1)
        acc2[...] = jnp.zeros_like(acc2)

    blk = c_ref[...]
    acc1[...] += jnp.sum(blk, axis=1, keepdims=True)
    acc2[...] += jnp.sum(blk * blk, axis=1, keepdims=True)

    @pl.when(i == NB - 1)
    def _():
        mu = acc1[...] / N
        var = acc2[...] / N - mu * mu
        inv = lax.rsqrt(var + 1e-5)
        sc = g_ref[...] * inv
        s_ref[...] = sc
        t_ref[...] = be_ref[...] - mu * sc


def _stats(conv, g, be):
    return pl.pallas_call(
        _stats_body,
        grid=(NB,),
        in_specs=[
            pl.BlockSpec((2, RB, DH), lambda i: (0, i, 0)),
            pl.BlockSpec((2, 1, DH), lambda i: (0, 0, 0)),
            pl.BlockSpec((2, 1, DH), lambda i: (0, 0, 0)),
        ],
        out_specs=[
            pl.BlockSpec((2, 1, DH), lambda i: (0, 0, 0)),
            pl.BlockSpec((2, 1, DH), lambda i: (0, 0, 0)),
        ],
        out_shape=[
            jax.ShapeDtypeStruct((2, 1, DH), jnp.float32),
            jax.ShapeDtypeStruct((2, 1, DH), jnp.float32),
        ],
        scratch_shapes=[
            pltpu.VMEM((2, 1, DH), jnp.float32),
            pltpu.VMEM((2, 1, DH), jnp.float32),
        ],
    )(conv, g, be)


def _pool_body(x_ref, c0_ref, s0_ref, t0_ref, c1_ref, s1_ref, t1_ref,
               b_ref, wp0_ref, wp1_ref, wp2_ref, bsum_ref, o_ref,
               acc, cnt):
    i = pl.program_id(0)

    @pl.when(i == 0)
    def _():
        acc[...] = jnp.zeros_like(acc)
        cnt[...] = jnp.zeros_like(cnt)

    b = b_ref[0]
    onehot = (lax.broadcasted_iota(jnp.int32, (G, RB), 0)
              == jnp.broadcast_to(b, (G, RB))).astype(jnp.float32)
    cnt[...] += jnp.sum(onehot, axis=1, keepdims=True)

    dn = (((1,), (0,)), ((), ()))

    p0 = lax.dot_general(onehot, x_ref[...], dn,
                         preferred_element_type=jnp.float32)
    acc[...] += jnp.dot(p0, wp0_ref[...], preferred_element_type=jnp.float32)

    h1 = jnp.concatenate(
        [jnp.maximum(c0_ref[0] * s0_ref[0] + t0_ref[0], 0.0),
         jnp.maximum(c0_ref[1] * s0_ref[1] + t0_ref[1], 0.0)], axis=1)
    p1 = lax.dot_general(onehot, h1, dn, preferred_element_type=jnp.float32)
    acc[...] += jnp.dot(p1, wp1_ref[...], preferred_element_type=jnp.float32)

    h2 = jnp.concatenate(
        [jnp.maximum(c1_ref[0] * s1_ref[0] + t1_ref[0], 0.0),
         jnp.maximum(c1_ref[1] * s1_ref[1] + t1_ref[1], 0.0)], axis=1)
    p2 = lax.dot_general(onehot, h2, dn, preferred_element_type=jnp.float32)
    acc[...] += jnp.dot(p2, wp2_ref[...], preferred_element_type=jnp.float32)

    @pl.when(i == NB - 1)
    def _():
        counts = jnp.maximum(cnt[...], 1.0)
        logits = acc[...] / counts + bsum_ref[...]
        o_ref[...] = 1.0 / (1.0 + jnp.exp(-logits))


def _pool(x, conv0, s0, t0, conv1, s1, t1, batch3, wp0, wp1, wp2, bsum):
    half = pl.BlockSpec((2, RB, DH), lambda i: (0, i, 0))
    stat = pl.BlockSpec((2, 1, DH), lambda i: (0, 0, 0))
    return pl.pallas_call(
        _pool_body,
        grid=(NB,),
        in_specs=[
            pl.BlockSpec((RB, D), lambda i: (i, 0)),
            half, stat, stat, half, stat, stat,
            pl.BlockSpec((1, 1, RB), lambda i: (i, 0, 0)),
            pl.BlockSpec((D, D_OUT), lambda i: (0, 0)),
            pl.BlockSpec((D, D_OUT), lambda i: (0, 0)),
            pl.BlockSpec((D, D_OUT), lambda i: (0, 0)),
            pl.BlockSpec((1, D_OUT), lambda i: (0, 0)),
        ],
        out_specs=pl.BlockSpec((G, D_OUT), lambda i: (0, 0)),
        out_shape=jax.ShapeDtypeStruct((G, D_OUT), jnp.float32),
        scratch_shapes=[
            pltpu.VMEM((G, D_OUT), jnp.float32),
            pltpu.VMEM((G, 1), jnp.float32),
        ],
    )(x, conv0, s0, t0, conv1, s1, t1, batch3, wp0, wp1, wp2, bsum)


def _msgpass_body(table, src_hbm, dst_hbm, ew_hbm, out_hbm,
                  sidx_v, didx_v, ew_v, rows_v, zbuf, acc, sem):
    cid = lax.axis_index("c")
    sid = lax.axis_index("s")
    row_off = cid * N

    def _zrow(r, _):
        for j in range(8):
            zbuf[r, pl.ds(j * 16, 16)] = jnp.zeros((16,), jnp.float32)
        return _
    lax.fori_loop(0, ZROWS, _zrow, 0)
    for q in range(RPT // ZROWS):
        pltpu.sync_copy(zbuf, acc.at[pl.ds(sid * RPT + q * ZROWS, ZROWS)])
    plsc.subcore_barrier()

    def _chunk(k, _):
        base = sid * EPT + k * CHUNK
        pltpu.sync_copy(src_hbm.at[pl.ds(base, CHUNK)], sidx_v)
        pltpu.sync_copy(dst_hbm.at[pl.ds(base, CHUNK)], didx_v)
        pltpu.sync_copy(ew_hbm.at[pl.ds(base, CHUNK)], ew_v)
        for t in range(CHUNK // 16):
            sl = pl.ds(t * 16, 16)
            sidx_v[sl] = sidx_v[sl] + row_off
        pltpu.async_copy(table.at[sidx_v], rows_v, sem).wait()

        def _edge16(e16, _):
            wv = ew_v[pl.ds(e16 * 16, 16)]
            for i in range(16):
                w = jnp.full((16,), wv[i], jnp.float32)
                e = e16 * 16 + i
                for j in range(8):
                    sl = pl.ds(j * 16, 16)
                    rows_v[e, sl] = rows_v[e, sl] * w
            return _
        lax.fori_loop(0, CHUNK // 16, _edge16, 0)

        pltpu.sync_copy(rows_v, acc.at[didx_v], add=True)
        return _
    lax.fori_loop(0, NCHUNK, _chunk, 0)

    plsc.subcore_barrier()
    pltpu.sync_copy(acc.at[pl.ds(sid * RPT, RPT)],
                    out_hbm.at[pl.ds(cid * NPAD + sid * RPT, RPT)])


_msgpass = functools.partial(
    pl.kernel,
    out_type=jax.ShapeDtypeStruct((2 * NPAD, DH), jnp.float32),
    mesh=plsc.VectorSubcoreMesh(core_axis_name="c", subcore_axis_name="s"),
    scratch_types=[
        pltpu.VMEM((CHUNK,), jnp.int32),
        pltpu.VMEM((CHUNK,), jnp.int32),
        pltpu.VMEM((CHUNK,), jnp.float32),
        pltpu.VMEM((CHUNK, DH), jnp.float32),
        pltpu.VMEM((ZROWS, DH), jnp.float32),
        pltpu.VMEM_SHARED((NPAD, DH), jnp.float32),
        pltpu.SemaphoreType.DMA,
    ],
)(_msgpass_body)


def _conv(hl, src, dst, ew):
    table = hl.reshape(2 * N, DH)
    out = _msgpass(table, src, dst, ew)
    return out.reshape(2, NPAD, DH)[:, :N, :]


def kernel(x, edge_index, edge_weight, batch,
           W0, b0, W1, b1, g0, be0, g1, be1,
           Wp0, bp0, Wp1, bp1, Wp2, bp2):
    src = edge_index[0]
    dst = edge_index[1]
    g0r = g0.reshape(2, 1, DH)
    be0r = be0.reshape(2, 1, DH)
    g1r = g1.reshape(2, 1, DH)
    be1r = be1.reshape(2, 1, DH)
    batch3 = batch.reshape(NB, 1, RB)
    bsum = (bp0 + bp1 + bp2).reshape(1, D_OUT)

    hl0 = _mm0(x, W0)
    conv0 = _conv(hl0, src, dst, ew=edge_weight)
    s0, t0 = _stats(conv0, g0r, be0r)
    hl1 = _mm1(conv0, s0, t0, W1)
    conv1 = _conv(hl1, src, dst, ew=edge_weight)
    s1, t1 = _stats(conv1, g1r, be1r)
    return _pool(x, conv0, s0, t0, conv1, s1, t1, batch3,
                 Wp0, Wp1, Wp2, bsum)


# speedup vs baseline: 6.9747x; 2.4292x over previous
import functools

import jax
import jax.numpy as jnp
from jax import lax
from jax.experimental import pallas as pl
from jax.experimental.pallas import tpu as pltpu
from jax.experimental.pallas import tpu_sc as plsc

N = 10000
E = 160000
D = 256
DH = 128
D_OUT = 128
G = 64

NB = 10
RB = N // NB

NTILES = 16
EPT = E // NTILES
CHUNK = 80
NCHUNK = EPT // CHUNK
NPAD = 10240
RPT = NPAD // NTILES
ZROWS = 128


def _mm0_body(x_ref, w_ref, o_ref):
    res = jnp.dot(x_ref[...], w_ref[...], preferred_element_type=jnp.float32)
    o_ref[0, :, :] = res[:, :DH]
    o_ref[1, :, :] = res[:, DH:]


def _mm0(x, w):
    return pl.pallas_call(
        _mm0_body,
        grid=(NB,),
        in_specs=[
            pl.BlockSpec((RB, D), lambda i: (i, 0)),
            pl.BlockSpec((D, D), lambda i: (0, 0)),
        ],
        out_specs=pl.BlockSpec((2, RB, DH), lambda i: (0, i, 0)),
        out_shape=jax.ShapeDtypeStruct((2, N, DH), jnp.float32),
    )(x, w)


def _mm1_body(c_ref, s_ref, t_ref, w_ref, o_ref):
    h_lo = jnp.maximum(c_ref[0] * s_ref[0] + t_ref[0], 0.0)
    h_hi = jnp.maximum(c_ref[1] * s_ref[1] + t_ref[1], 0.0)
    h = jnp.concatenate([h_lo, h_hi], axis=1)
    res = jnp.dot(h, w_ref[...], preferred_element_type=jnp.float32)
    o_ref[0, :, :] = res[:, :DH]
    o_ref[1, :, :] = res[:, DH:]


def _mm1(conv, scale, shift, w):
    return pl.pallas_call(
        _mm1_body,
        grid=(NB,),
        in_specs=[
            pl.BlockSpec((2, RB, DH), lambda i: (0, i, 0)),
            pl.BlockSpec((2, 1, DH), lambda i: (0, 0, 0)),
            pl.BlockSpec((2, 1, DH), lambda i: (0, 0, 0)),
            pl.BlockSpec((D, D), lambda i: (0, 0)),
        ],
        out_specs=pl.BlockSpec((2, RB, DH), lambda i: (0, i, 0)),
        out_shape=jax.ShapeDtypeStruct((2, N, DH), jnp.float32),
    )(conv, scale, shift, w)


def _stats_body(c_ref, g_ref, be_ref, s_ref, t_ref, acc1, acc2):
    i = pl.program_id(0)

    @pl.when(i == 0)
    def _():
        acc1[...] = jnp.zeros_like(acc1)
        acc2[...] = jnp.zeros_like(acc2)

    blk = c_ref[...]
    acc1[...] += jnp.sum(blk, axis=1, keepdims=True)
    acc2[...] += jnp.sum(blk * blk, axis=1, keepdims=True)

    @pl.when(i == NB - 1)
    def _():
        mu = acc1[...] / N
        var = acc2[...] / N - mu * mu
        inv = lax.rsqrt(var + 1e-5)
        sc = g_ref[...] * inv
        s_ref[...] = sc
        t_ref[...] = be_ref[...] - mu * sc


def _stats(conv, g, be):
    return pl.pallas_call(
        _stats_body,
        grid=(NB,),
        in_specs=[
            pl.BlockSpec((2, RB, DH), lambda i: (0, i, 0)),
            pl.BlockSpec((2, 1, DH), lambda i: (0, 0, 0)),
            pl.BlockSpec((2, 1, DH), lambda i: (0, 0, 0)),
        ],
        out_specs=[
            pl.BlockSpec((2, 1, DH), lambda i: (0, 0, 0)),
            pl.BlockSpec((2, 1, DH), lambda i: (0, 0, 0)),
        ],
        out_shape=[
            jax.ShapeDtypeStruct((2, 1, DH), jnp.float32),
            jax.ShapeDtypeStruct((2, 1, DH), jnp.float32),
        ],
        scratch_shapes=[
            pltpu.VMEM((2, 1, DH), jnp.float32),
            pltpu.VMEM((2, 1, DH), jnp.float32),
        ],
    )(conv, g, be)


def _pool_body(x_ref, c0_ref, s0_ref, t0_ref, c1_ref, s1_ref, t1_ref,
               b_ref, wp0_ref, wp1_ref, wp2_ref, bsum_ref, o_ref,
               acc, cnt):
    i = pl.program_id(0)

    @pl.when(i == 0)
    def _():
        acc[...] = jnp.zeros_like(acc)
        cnt[...] = jnp.zeros_like(cnt)

    b = b_ref[0]
    onehot = (lax.broadcasted_iota(jnp.int32, (G, RB), 0)
              == jnp.broadcast_to(b, (G, RB))).astype(jnp.float32)
    cnt[...] += jnp.sum(onehot, axis=1, keepdims=True)

    dn = (((1,), (0,)), ((), ()))

    p0 = lax.dot_general(onehot, x_ref[...], dn,
                         preferred_element_type=jnp.float32)
    acc[...] += jnp.dot(p0, wp0_ref[...], preferred_element_type=jnp.float32)

    h1 = jnp.concatenate(
        [jnp.maximum(c0_ref[0] * s0_ref[0] + t0_ref[0], 0.0),
         jnp.maximum(c0_ref[1] * s0_ref[1] + t0_ref[1], 0.0)], axis=1)
    p1 = lax.dot_general(onehot, h1, dn, preferred_element_type=jnp.float32)
    acc[...] += jnp.dot(p1, wp1_ref[...], preferred_element_type=jnp.float32)

    h2 = jnp.concatenate(
        [jnp.maximum(c1_ref[0] * s1_ref[0] + t1_ref[0], 0.0),
         jnp.maximum(c1_ref[1] * s1_ref[1] + t1_ref[1], 0.0)], axis=1)
    p2 = lax.dot_general(onehot, h2, dn, preferred_element_type=jnp.float32)
    acc[...] += jnp.dot(p2, wp2_ref[...], preferred_element_type=jnp.float32)

    @pl.when(i == NB - 1)
    def _():
        counts = jnp.maximum(cnt[...], 1.0)
        logits = acc[...] / counts + bsum_ref[...]
        o_ref[...] = 1.0 / (1.0 + jnp.exp(-logits))


def _pool(x, conv0, s0, t0, conv1, s1, t1, batch3, wp0, wp1, wp2, bsum):
    half = pl.BlockSpec((2, RB, DH), lambda i: (0, i, 0))
    stat = pl.BlockSpec((2, 1, DH), lambda i: (0, 0, 0))
    return pl.pallas_call(
        _pool_body,
        grid=(NB,),
        in_specs=[
            pl.BlockSpec((RB, D), lambda i: (i, 0)),
            half, stat, stat, half, stat, stat,
            pl.BlockSpec((1, 1, RB), lambda i: (i, 0, 0)),
            pl.BlockSpec((D, D_OUT), lambda i: (0, 0)),
            pl.BlockSpec((D, D_OUT), lambda i: (0, 0)),
            pl.BlockSpec((D, D_OUT), lambda i: (0, 0)),
            pl.BlockSpec((1, D_OUT), lambda i: (0, 0)),
        ],
        out_specs=pl.BlockSpec((G, D_OUT), lambda i: (0, 0)),
        out_shape=jax.ShapeDtypeStruct((G, D_OUT), jnp.float32),
        scratch_shapes=[
            pltpu.VMEM((G, D_OUT), jnp.float32),
            pltpu.VMEM((G, 1), jnp.float32),
        ],
    )(x, conv0, s0, t0, conv1, s1, t1, batch3, wp0, wp1, wp2, bsum)


def _msgpass_body(table, src_hbm, dst_hbm, ew_hbm, out_hbm,
                  sidx_v, ew_v, didx0, didx1, rows0, rows1, acc,
                  semg0, semg1, semi0, semi1):
    cid = lax.axis_index("c")
    sid = lax.axis_index("s")
    row_off = cid * N
    ebase = sid * EPT

    pltpu.async_copy(src_hbm.at[pl.ds(ebase, EPT)], sidx_v, semi0)
    pltpu.async_copy(ew_hbm.at[pl.ds(ebase, EPT)], ew_v, semi1)

    def _zrow(r, c):
        for j in range(8):
            sl = pl.ds(j * 16, 16)
            rows0[r, sl] = jnp.zeros((16,), jnp.float32)
            rows1[r, sl] = jnp.zeros((16,), jnp.float32)
        return c
    lax.fori_loop(0, CHUNK, _zrow, 0)
    for q in range(RPT // (2 * CHUNK)):
        pltpu.sync_copy(rows0, acc.at[pl.ds(sid * RPT + 2 * q * CHUNK, CHUNK)])
        pltpu.sync_copy(rows1,
                        acc.at[pl.ds(sid * RPT + (2 * q + 1) * CHUNK, CHUNK)])

    pltpu.make_async_copy(src_hbm.at[pl.ds(ebase, EPT)], sidx_v, semi0).wait()
    pltpu.make_async_copy(ew_hbm.at[pl.ds(ebase, EPT)], ew_v, semi1).wait()

    def _off(t, c):
        sl = pl.ds(t * 16, 16)
        sidx_v[sl] = sidx_v[sl] + row_off
        return c
    lax.fori_loop(0, EPT // 16, _off, 0)
    plsc.subcore_barrier()

    def _gather(k, rows, sem):
        pltpu.async_copy(table.at[sidx_v.at[pl.ds(k * CHUNK, CHUNK)]],
                         rows, sem)

    def _gwait(k, rows, sem):
        pltpu.make_async_copy(table.at[sidx_v.at[pl.ds(k * CHUNK, CHUNK)]],
                              rows, sem).wait()

    def _dpre(k, didx, sem):
        pltpu.async_copy(dst_hbm.at[pl.ds(ebase + k * CHUNK, CHUNK)], didx, sem)

    def _dwait(k, didx, sem):
        pltpu.make_async_copy(dst_hbm.at[pl.ds(ebase + k * CHUNK, CHUNK)],
                              didx, sem).wait()

    def _mul(k, rows):
        def _edge16(e16, c):
            wv = ew_v[pl.ds(k * CHUNK + e16 * 16, 16)]
            for i in range(16):
                w = jnp.full((16,), wv[i], jnp.float32)
                e = e16 * 16 + i
                for j in range(8):
                    sl = pl.ds(j * 16, 16)
                    rows[e, sl] = rows[e, sl] * w
            return c
        lax.fori_loop(0, CHUNK // 16, _edge16, 0)

    _dpre(0, didx0, semi0)
    _gather(0, rows0, semg0)
    _dpre(1, didx1, semi1)

    def _chunk2(k2, carry):
        k = k2 * 2
        _gather(k + 1, rows1, semg1)
        _gwait(k, rows0, semg0)
        _mul(k, rows0)
        _dwait(k, didx0, semi0)
        pltpu.sync_copy(rows0, acc.at[didx0], add=True)
        _dpre(k + 2, didx0, semi0)

        @pl.when(k + 2 < NCHUNK)
        def _():
            _gather(k + 2, rows0, semg0)
        _gwait(k + 1, rows1, semg1)
        _mul(k + 1, rows1)
        _dwait(k + 1, didx1, semi1)
        pltpu.sync_copy(rows1, acc.at[didx1], add=True)

        @pl.when(k + 3 < NCHUNK)
        def _():
            _dpre(k + 3, didx1, semi1)
        return carry
    lax.fori_loop(0, NCHUNK // 2, _chunk2, 0)

    k = NCHUNK - 1
    _gwait(k, rows0, semg0)
    _mul(k, rows0)
    _dwait(k, didx0, semi0)
    pltpu.sync_copy(rows0, acc.at[didx0], add=True)

    plsc.subcore_barrier()
    pltpu.sync_copy(acc.at[pl.ds(sid * RPT, RPT)],
                    out_hbm.at[pl.ds(cid * NPAD + sid * RPT, RPT)])


_msgpass = functools.partial(
    pl.kernel,
    out_type=jax.ShapeDtypeStruct((2 * NPAD, DH), jnp.float32),
    mesh=plsc.VectorSubcoreMesh(core_axis_name="c", subcore_axis_name="s"),
    scratch_types=[
        pltpu.VMEM((EPT,), jnp.int32),
        pltpu.VMEM((EPT,), jnp.float32),
        pltpu.VMEM((CHUNK,), jnp.int32),
        pltpu.VMEM((CHUNK,), jnp.int32),
        pltpu.VMEM((CHUNK, DH), jnp.float32),
        pltpu.VMEM((CHUNK, DH), jnp.float32),
        pltpu.VMEM_SHARED((NPAD, DH), jnp.float32),
        pltpu.SemaphoreType.DMA,
        pltpu.SemaphoreType.DMA,
        pltpu.SemaphoreType.DMA,
        pltpu.SemaphoreType.DMA,
    ],
)(_msgpass_body)


def _conv(hl, src, dst2, ew):
    table = hl.reshape(2 * N, DH)
    out = _msgpass(table, src, dst2, ew)
    return out.reshape(2, NPAD, DH)[:, :N, :]


def kernel(x, edge_index, edge_weight, batch,
           W0, b0, W1, b1, g0, be0, g1, be1,
           Wp0, bp0, Wp1, bp1, Wp2, bp2):
    src = edge_index[0]
    dst2 = edge_index[1]
    g0r = g0.reshape(2, 1, DH)
    be0r = be0.reshape(2, 1, DH)
    g1r = g1.reshape(2, 1, DH)
    be1r = be1.reshape(2, 1, DH)
    batch3 = batch.reshape(NB, 1, RB)
    bsum = (bp0 + bp1 + bp2).reshape(1, D_OUT)

    hl0 = _mm0(x, W0)
    conv0 = _conv(hl0, src, dst2, ew=edge_weight)
    s0, t0 = _stats(conv0, g0r, be0r)
    hl1 = _mm1(conv0, s0, t0, W1)
    conv1 = _conv(hl1, src, dst2, ew=edge_weight)
    s1, t1 = _stats(conv1, g1r, be1r)
    return _pool(x, conv0, s0, t0, conv1, s1, t1, batch3,
                 Wp0, Wp1, Wp2, bsum)


# speedup vs baseline: 7.7348x; 1.1090x over previous
import functools

import jax
import jax.numpy as jnp
from jax import lax
from jax.experimental import pallas as pl
from jax.experimental.pallas import tpu as pltpu
from jax.experimental.pallas import tpu_sc as plsc

N = 10000
E = 160000
D = 256
DH = 128
D_OUT = 128
G = 64

NB = 10
RB = N // NB

NTILES = 16
EPT = E // NTILES
CHUNK = 80
NCHUNK = EPT // CHUNK
NPAD = 10240
RPT = NPAD // NTILES
ZROWS = 128


def _mm0_body(x_ref, w_ref, o_ref):
    res = jnp.dot(x_ref[...], w_ref[...], preferred_element_type=jnp.float32)
    o_ref[0, :, :] = res[:, :DH]
    o_ref[1, :, :] = res[:, DH:]


def _mm0(x, w):
    return pl.pallas_call(
        _mm0_body,
        grid=(NB,),
        in_specs=[
            pl.BlockSpec((RB, D), lambda i: (i, 0)),
            pl.BlockSpec((D, D), lambda i: (0, 0)),
        ],
        out_specs=pl.BlockSpec((2, RB, DH), lambda i: (0, i, 0)),
        out_shape=jax.ShapeDtypeStruct((2, N, DH), jnp.float32),
    )(x, w)


def _mm1_body(c_ref, s_ref, t_ref, w_ref, o_ref):
    h_lo = jnp.maximum(c_ref[0] * s_ref[0] + t_ref[0], 0.0)
    h_hi = jnp.maximum(c_ref[1] * s_ref[1] + t_ref[1], 0.0)
    h = jnp.concatenate([h_lo, h_hi], axis=1)
    res = jnp.dot(h, w_ref[...], preferred_element_type=jnp.float32)
    o_ref[0, :, :] = res[:, :DH]
    o_ref[1, :, :] = res[:, DH:]


def _mm1(conv, scale, shift, w):
    return pl.pallas_call(
        _mm1_body,
        grid=(NB,),
        in_specs=[
            pl.BlockSpec((2, RB, DH), lambda i: (0, i, 0)),
            pl.BlockSpec((2, 1, DH), lambda i: (0, 0, 0)),
            pl.BlockSpec((2, 1, DH), lambda i: (0, 0, 0)),
            pl.BlockSpec((D, D), lambda i: (0, 0)),
        ],
        out_specs=pl.BlockSpec((2, RB, DH), lambda i: (0, i, 0)),
        out_shape=jax.ShapeDtypeStruct((2, N, DH), jnp.float32),
    )(conv, scale, shift, w)


def _stats_body(c_ref, g_ref, be_ref, s_ref, t_ref, acc1, acc2):
    i = pl.program_id(0)

    @pl.when(i == 0)
    def _():
        acc1[...] = jnp.zeros_like(acc1)
        acc2[...] = jnp.zeros_like(acc2)

    blk = c_ref[...]
    acc1[...] += jnp.sum(blk, axis=1, keepdims=True)
    acc2[...] += jnp.sum(blk * blk, axis=1, keepdims=True)

    @pl.when(i == NB - 1)
    def _():
        mu = acc1[...] / N
        var = acc2[...] / N - mu * mu
        inv = lax.rsqrt(var + 1e-5)
        sc = g_ref[...] * inv
        s_ref[...] = sc
        t_ref[...] = be_ref[...] - mu * sc


def _stats(conv, g, be):
    return pl.pallas_call(
        _stats_body,
        grid=(NB,),
        in_specs=[
            pl.BlockSpec((2, RB, DH), lambda i: (0, i, 0)),
            pl.BlockSpec((2, 1, DH), lambda i: (0, 0, 0)),
            pl.BlockSpec((2, 1, DH), lambda i: (0, 0, 0)),
        ],
        out_specs=[
            pl.BlockSpec((2, 1, DH), lambda i: (0, 0, 0)),
            pl.BlockSpec((2, 1, DH), lambda i: (0, 0, 0)),
        ],
        out_shape=[
            jax.ShapeDtypeStruct((2, 1, DH), jnp.float32),
            jax.ShapeDtypeStruct((2, 1, DH), jnp.float32),
        ],
        scratch_shapes=[
            pltpu.VMEM((2, 1, DH), jnp.float32),
            pltpu.VMEM((2, 1, DH), jnp.float32),
        ],
    )(conv, g, be)


def _pool_body(x_ref, c0_ref, s0_ref, t0_ref, c1_ref, s1_ref, t1_ref,
               b_ref, wp0_ref, wp1_ref, wp2_ref, bsum_ref, o_ref,
               acc, cnt):
    i = pl.program_id(0)

    @pl.when(i == 0)
    def _():
        acc[...] = jnp.zeros_like(acc)
        cnt[...] = jnp.zeros_like(cnt)

    b = b_ref[0]
    onehot = (lax.broadcasted_iota(jnp.int32, (G, RB), 0)
              == jnp.broadcast_to(b, (G, RB))).astype(jnp.float32)
    cnt[...] += jnp.sum(onehot, axis=1, keepdims=True)

    dn = (((1,), (0,)), ((), ()))

    p0 = lax.dot_general(onehot, x_ref[...], dn,
                         preferred_element_type=jnp.float32)
    acc[...] += jnp.dot(p0, wp0_ref[...], preferred_element_type=jnp.float32)

    h1 = jnp.concatenate(
        [jnp.maximum(c0_ref[0] * s0_ref[0] + t0_ref[0], 0.0),
         jnp.maximum(c0_ref[1] * s0_ref[1] + t0_ref[1], 0.0)], axis=1)
    p1 = lax.dot_general(onehot, h1, dn, preferred_element_type=jnp.float32)
    acc[...] += jnp.dot(p1, wp1_ref[...], preferred_element_type=jnp.float32)

    h2 = jnp.concatenate(
        [jnp.maximum(c1_ref[0] * s1_ref[0] + t1_ref[0], 0.0),
         jnp.maximum(c1_ref[1] * s1_ref[1] + t1_ref[1], 0.0)], axis=1)
    p2 = lax.dot_general(onehot, h2, dn, preferred_element_type=jnp.float32)
    acc[...] += jnp.dot(p2, wp2_ref[...], preferred_element_type=jnp.float32)

    @pl.when(i == NB - 1)
    def _():
        counts = jnp.maximum(cnt[...], 1.0)
        logits = acc[...] / counts + bsum_ref[...]
        o_ref[...] = 1.0 / (1.0 + jnp.exp(-logits))


def _pool(x, conv0, s0, t0, conv1, s1, t1, batch3, wp0, wp1, wp2, bsum):
    half = pl.BlockSpec((2, RB, DH), lambda i: (0, i, 0))
    stat = pl.BlockSpec((2, 1, DH), lambda i: (0, 0, 0))
    return pl.pallas_call(
        _pool_body,
        grid=(NB,),
        in_specs=[
            pl.BlockSpec((RB, D), lambda i: (i, 0)),
            half, stat, stat, half, stat, stat,
            pl.BlockSpec((1, 1, RB), lambda i: (i, 0, 0)),
            pl.BlockSpec((D, D_OUT), lambda i: (0, 0)),
            pl.BlockSpec((D, D_OUT), lambda i: (0, 0)),
            pl.BlockSpec((D, D_OUT), lambda i: (0, 0)),
            pl.BlockSpec((1, D_OUT), lambda i: (0, 0)),
        ],
        out_specs=pl.BlockSpec((G, D_OUT), lambda i: (0, 0)),
        out_shape=jax.ShapeDtypeStruct((G, D_OUT), jnp.float32),
        scratch_shapes=[
            pltpu.VMEM((G, D_OUT), jnp.float32),
            pltpu.VMEM((G, 1), jnp.float32),
        ],
    )(x, conv0, s0, t0, conv1, s1, t1, batch3, wp0, wp1, wp2, bsum)


def _msgpass_body(table, src_hbm, dst_hbm, ew_hbm, out_hbm,
                  sidx_v, didx0, didx1, didx2, ew0, ew1, ew2,
                  rows0, rows1, rows2, acc,
                  semg0, semg1, semg2, sems0, sems1, sems2,
                  semi0, semi1, semi2):
    cid = lax.axis_index("c")
    sid = lax.axis_index("s")
    row_off = cid * N
    ebase = sid * EPT

    pltpu.async_copy(src_hbm.at[pl.ds(ebase, EPT)], sidx_v, semi0)

    def _zrow(r, c):
        for j in range(8):
            sl = pl.ds(j * 16, 16)
            rows0[r, sl] = jnp.zeros((16,), jnp.float32)
            rows1[r, sl] = jnp.zeros((16,), jnp.float32)
        return c
    lax.fori_loop(0, CHUNK, _zrow, 0)
    for q in range(RPT // (2 * CHUNK)):
        pltpu.sync_copy(rows0, acc.at[pl.ds(sid * RPT + 2 * q * CHUNK, CHUNK)])
        pltpu.sync_copy(rows1,
                        acc.at[pl.ds(sid * RPT + (2 * q + 1) * CHUNK, CHUNK)])

    pltpu.make_async_copy(src_hbm.at[pl.ds(ebase, EPT)], sidx_v, semi0).wait()

    def _off(t, c):
        sl = pl.ds(t * 16, 16)
        sidx_v[sl] = sidx_v[sl] + row_off
        return c
    lax.fori_loop(0, EPT // 16, _off, 0)
    plsc.subcore_barrier()

    def _gather(k, rows, sem):
        pltpu.async_copy(table.at[sidx_v.at[pl.ds(k * CHUNK, CHUNK)]],
                         rows, sem)

    def _gwait(k, rows, sem):
        pltpu.make_async_copy(table.at[sidx_v.at[pl.ds(k * CHUNK, CHUNK)]],
                              rows, sem).wait()

    def _dpre(k, didx, ew, sem):
        pltpu.async_copy(dst_hbm.at[pl.ds(ebase + k * CHUNK, CHUNK)],
                         didx, sem)
        pltpu.async_copy(ew_hbm.at[pl.ds(ebase + k * CHUNK, CHUNK)], ew, sem)

    def _dwait(k, didx, ew, sem):
        pltpu.make_async_copy(dst_hbm.at[pl.ds(ebase + k * CHUNK, CHUNK)],
                              didx, sem).wait()
        pltpu.make_async_copy(ew_hbm.at[pl.ds(ebase + k * CHUNK, CHUNK)],
                              ew, sem).wait()

    def _swait(rows, didx, sem):
        pltpu.make_async_copy(rows, acc.at[didx], sem).wait()

    def _mul(ew, rows):
        def _edge16(e16, c):
            wv = ew[pl.ds(e16 * 16, 16)]
            for i in range(16):
                w = jnp.full((16,), wv[i], jnp.float32)
                e = e16 * 16 + i
                for j in range(8):
                    sl = pl.ds(j * 16, 16)
                    rows[e, sl] = rows[e, sl] * w
            return c
        lax.fori_loop(0, CHUNK // 16, _edge16, 0)

    def _proc(k, rows, didx, ew, semg, semi, sems):
        _gwait(k, rows, semg)
        _dwait(k, didx, ew, semi)
        _mul(ew, rows)
        pltpu.async_copy(rows, acc.at[didx], sems, add=True)

    _dpre(0, didx0, ew0, semi0)
    _dpre(1, didx1, ew1, semi1)
    _gather(0, rows0, semg0)
    _gather(1, rows1, semg1)

    def _chunk3(j, carry):
        k = j * 3
        _proc(k, rows0, didx0, ew0, semg0, semi0, sems0)

        @pl.when(k > 0)
        def _():
            _swait(rows2, didx2, sems2)
        _dpre(k + 2, didx2, ew2, semi2)
        _gather(k + 2, rows2, semg2)

        _proc(k + 1, rows1, didx1, ew1, semg1, semi1, sems1)

        _swait(rows0, didx0, sems0)
        _dpre(k + 3, didx0, ew0, semi0)
        _gather(k + 3, rows0, semg0)

        _proc(k + 2, rows2, didx2, ew2, semg2, semi2, sems2)

        _swait(rows1, didx1, sems1)
        _dpre(k + 4, didx1, ew1, semi1)
        _gather(k + 4, rows1, semg1)
        return carry
    lax.fori_loop(0, NCHUNK // 3, _chunk3, 0)

    _proc(NCHUNK - 2, rows0, didx0, ew0, semg0, semi0, sems0)
    _proc(NCHUNK - 1, rows1, didx1, ew1, semg1, semi1, sems1)
    _swait(rows2, didx2, sems2)
    _swait(rows0, didx0, sems0)
    _swait(rows1, didx1, sems1)

    plsc.subcore_barrier()
    pltpu.sync_copy(acc.at[pl.ds(sid * RPT, RPT)],
                    out_hbm.at[pl.ds(cid * NPAD + sid * RPT, RPT)])


_msgpass = functools.partial(
    pl.kernel,
    out_type=jax.ShapeDtypeStruct((2 * NPAD, DH), jnp.float32),
    mesh=plsc.VectorSubcoreMesh(core_axis_name="c", subcore_axis_name="s"),
    scratch_types=[
        pltpu.VMEM((EPT,), jnp.int32),
        pltpu.VMEM((CHUNK,), jnp.int32),
        pltpu.VMEM((CHUNK,), jnp.int32),
        pltpu.VMEM((CHUNK,), jnp.int32),
        pltpu.VMEM((CHUNK,), jnp.float32),
        pltpu.VMEM((CHUNK,), jnp.float32),
        pltpu.VMEM((CHUNK,), jnp.float32),
        pltpu.VMEM((CHUNK, DH), jnp.float32),
        pltpu.VMEM((CHUNK, DH), jnp.float32),
        pltpu.VMEM((CHUNK, DH), jnp.float32),
        pltpu.VMEM_SHARED((NPAD, DH), jnp.float32),
    ] + [pltpu.SemaphoreType.DMA] * 9,
)(_msgpass_body)


def _conv(hl, src, dst2, ew):
    table = hl.reshape(2 * N, DH)
    out = _msgpass(table, src, dst2, ew)
    return out.reshape(2, NPAD, DH)[:, :N, :]


def kernel(x, edge_index, edge_weight, batch,
           W0, b0, W1, b1, g0, be0, g1, be1,
           Wp0, bp0, Wp1, bp1, Wp2, bp2):
    src = edge_index[0]
    dst2 = edge_index[1]
    g0r = g0.reshape(2, 1, DH)
    be0r = be0.reshape(2, 1, DH)
    g1r = g1.reshape(2, 1, DH)
    be1r = be1.reshape(2, 1, DH)
    batch3 = batch.reshape(NB, 1, RB)
    bsum = (bp0 + bp1 + bp2).reshape(1, D_OUT)

    hl0 = _mm0(x, W0)
    conv0 = _conv(hl0, src, dst2, ew=edge_weight)
    s0, t0 = _stats(conv0, g0r, be0r)
    hl1 = _mm1(conv0, s0, t0, W1)
    conv1 = _conv(hl1, src, dst2, ew=edge_weight)
    s1, t1 = _stats(conv1, g1r, be1r)
    return _pool(x, conv0, s0, t0, conv1, s1, t1, batch3,
                 Wp0, Wp1, Wp2, bsum)


# speedup vs baseline: 8.0739x; 1.0438x over previous
import functools

import jax
import jax.numpy as jnp
from jax import lax
from jax.experimental import pallas as pl
from jax.experimental.pallas import tpu as pltpu
from jax.experimental.pallas import tpu_sc as plsc

N = 10000
E = 160000
D = 256
DH = 128
D_OUT = 128
G = 64

NB = 10
RB = N // NB

NTILES = 16
EPT = E // NTILES
CHUNK = 80
NCHUNK = EPT // CHUNK
NPAD = 10240
RPT = NPAD // NTILES


def _mm0_body(x_ref, w_ref, o_ref):
    res = jnp.dot(x_ref[...], w_ref[...], preferred_element_type=jnp.float32)
    o_ref[0, :, :] = res[:, :DH]
    o_ref[1, :, :] = res[:, DH:]


def _mm0(x, w):
    return pl.pallas_call(
        _mm0_body,
        grid=(NB,),
        in_specs=[
            pl.BlockSpec((RB, D), lambda i: (i, 0)),
            pl.BlockSpec((D, D), lambda i: (0, 0)),
        ],
        out_specs=pl.BlockSpec((2, RB, DH), lambda i: (0, i, 0)),
        out_shape=jax.ShapeDtypeStruct((2, N, DH), jnp.float32),
    )(x, w)


def _mm1_body(c_ref, s_ref, t_ref, w_ref, o_ref):
    h_lo = jnp.maximum(c_ref[0] * s_ref[0] + t_ref[0], 0.0)
    h_hi = jnp.maximum(c_ref[1] * s_ref[1] + t_ref[1], 0.0)
    h = jnp.concatenate([h_lo, h_hi], axis=1)
    res = jnp.dot(h, w_ref[...], preferred_element_type=jnp.float32)
    o_ref[0, :, :] = res[:, :DH]
    o_ref[1, :, :] = res[:, DH:]


def _mm1(conv, scale, shift, w):
    return pl.pallas_call(
        _mm1_body,
        grid=(NB,),
        in_specs=[
            pl.BlockSpec((2, RB, DH), lambda i: (0, i, 0)),
            pl.BlockSpec((2, 1, DH), lambda i: (0, 0, 0)),
            pl.BlockSpec((2, 1, DH), lambda i: (0, 0, 0)),
            pl.BlockSpec((D, D), lambda i: (0, 0)),
        ],
        out_specs=pl.BlockSpec((2, RB, DH), lambda i: (0, i, 0)),
        out_shape=jax.ShapeDtypeStruct((2, N, DH), jnp.float32),
    )(conv, scale, shift, w)


def _stats_body(c_ref, g_ref, be_ref, s_ref, t_ref, acc1, acc2):
    i = pl.program_id(0)

    @pl.when(i == 0)
    def _():
        acc1[...] = jnp.zeros_like(acc1)
        acc2[...] = jnp.zeros_like(acc2)

    blk = c_ref[...]
    acc1[...] += jnp.sum(blk, axis=1, keepdims=True)
    acc2[...] += jnp.sum(blk * blk, axis=1, keepdims=True)

    @pl.when(i == NB - 1)
    def _():
        mu = acc1[...] / N
        var = acc2[...] / N - mu * mu
        inv = lax.rsqrt(var + 1e-5)
        sc = g_ref[...] * inv
        s_ref[...] = sc
        t_ref[...] = be_ref[...] - mu * sc


def _stats(conv, g, be):
    return pl.pallas_call(
        _stats_body,
        grid=(NB,),
        in_specs=[
            pl.BlockSpec((2, RB, DH), lambda i: (0, i, 0)),
            pl.BlockSpec((2, 1, DH), lambda i: (0, 0, 0)),
            pl.BlockSpec((2, 1, DH), lambda i: (0, 0, 0)),
        ],
        out_specs=[
            pl.BlockSpec((2, 1, DH), lambda i: (0, 0, 0)),
            pl.BlockSpec((2, 1, DH), lambda i: (0, 0, 0)),
        ],
        out_shape=[
            jax.ShapeDtypeStruct((2, 1, DH), jnp.float32),
            jax.ShapeDtypeStruct((2, 1, DH), jnp.float32),
        ],
        scratch_shapes=[
            pltpu.VMEM((2, 1, DH), jnp.float32),
            pltpu.VMEM((2, 1, DH), jnp.float32),
        ],
    )(conv, g, be)


def _pool_body(x_ref, c0_ref, s0_ref, t0_ref, c1_ref, s1_ref, t1_ref,
               b_ref, wp0_ref, wp1_ref, wp2_ref, bsum_ref, o_ref,
               acc, cnt):
    i = pl.program_id(0)

    @pl.when(i == 0)
    def _():
        acc[...] = jnp.zeros_like(acc)
        cnt[...] = jnp.zeros_like(cnt)

    b = b_ref[0]
    onehot = (lax.broadcasted_iota(jnp.int32, (G, RB), 0)
              == jnp.broadcast_to(b, (G, RB))).astype(jnp.float32)
    cnt[...] += jnp.sum(onehot, axis=1, keepdims=True)

    dn = (((1,), (0,)), ((), ()))

    p0 = lax.dot_general(onehot, x_ref[...], dn,
                         preferred_element_type=jnp.float32)
    acc[...] += jnp.dot(p0, wp0_ref[...], preferred_element_type=jnp.float32)

    h1 = jnp.concatenate(
        [jnp.maximum(c0_ref[0] * s0_ref[0] + t0_ref[0], 0.0),
         jnp.maximum(c0_ref[1] * s0_ref[1] + t0_ref[1], 0.0)], axis=1)
    p1 = lax.dot_general(onehot, h1, dn, preferred_element_type=jnp.float32)
    acc[...] += jnp.dot(p1, wp1_ref[...], preferred_element_type=jnp.float32)

    h2 = jnp.concatenate(
        [jnp.maximum(c1_ref[0] * s1_ref[0] + t1_ref[0], 0.0),
         jnp.maximum(c1_ref[1] * s1_ref[1] + t1_ref[1], 0.0)], axis=1)
    p2 = lax.dot_general(onehot, h2, dn, preferred_element_type=jnp.float32)
    acc[...] += jnp.dot(p2, wp2_ref[...], preferred_element_type=jnp.float32)

    @pl.when(i == NB - 1)
    def _():
        counts = jnp.maximum(cnt[...], 1.0)
        logits = acc[...] / counts + bsum_ref[...]
        o_ref[...] = 1.0 / (1.0 + jnp.exp(-logits))


def _pool(x, conv0, s0, t0, conv1, s1, t1, batch3, wp0, wp1, wp2, bsum):
    half = pl.BlockSpec((2, RB, DH), lambda i: (0, i, 0))
    stat = pl.BlockSpec((2, 1, DH), lambda i: (0, 0, 0))
    return pl.pallas_call(
        _pool_body,
        grid=(NB,),
        in_specs=[
            pl.BlockSpec((RB, D), lambda i: (i, 0)),
            half, stat, stat, half, stat, stat,
            pl.BlockSpec((1, 1, RB), lambda i: (i, 0, 0)),
            pl.BlockSpec((D, D_OUT), lambda i: (0, 0)),
            pl.BlockSpec((D, D_OUT), lambda i: (0, 0)),
            pl.BlockSpec((D, D_OUT), lambda i: (0, 0)),
            pl.BlockSpec((1, D_OUT), lambda i: (0, 0)),
        ],
        out_specs=pl.BlockSpec((G, D_OUT), lambda i: (0, 0)),
        out_shape=jax.ShapeDtypeStruct((G, D_OUT), jnp.float32),
        scratch_shapes=[
            pltpu.VMEM((G, D_OUT), jnp.float32),
            pltpu.VMEM((G, 1), jnp.float32),
        ],
    )(x, conv0, s0, t0, conv1, s1, t1, batch3, wp0, wp1, wp2, bsum)


def _msgpass_body(table, src_hbm, dst_hbm, ew_hbm, out_hbm,
                  sidx_v, didx0, didx1, didx2, ew0, ew1, ew2,
                  rows0, rows1, rows2, acc,
                  semg0, semg1, semg2, sems0, sems1, sems2,
                  semi0, semi1, semi2):
    cid = lax.axis_index("c")
    sid = lax.axis_index("s")
    row_off = cid * N
    ebase = sid * EPT

    pltpu.async_copy(src_hbm.at[pl.ds(ebase, EPT)], sidx_v, semi0)

    def _zrow(r, c):
        for j in range(8):
            sl = pl.ds(j * 16, 16)
            rows0[r, sl] = jnp.zeros((16,), jnp.float32)
            rows1[r, sl] = jnp.zeros((16,), jnp.float32)
        return c
    lax.fori_loop(0, CHUNK, _zrow, 0)
    for q in range(RPT // (2 * CHUNK)):
        pltpu.sync_copy(rows0, acc.at[pl.ds(sid * RPT + 2 * q * CHUNK, CHUNK)])
        pltpu.sync_copy(rows1,
                        acc.at[pl.ds(sid * RPT + (2 * q + 1) * CHUNK, CHUNK)])

    pltpu.make_async_copy(src_hbm.at[pl.ds(ebase, EPT)], sidx_v, semi0).wait()

    def _off(t, c):
        sl = pl.ds(t * 16, 16)
        sidx_v[sl] = sidx_v[sl] + row_off
        return c
    lax.fori_loop(0, EPT // 16, _off, 0)
    plsc.subcore_barrier()

    def _gather(k, rows, sem):
        pltpu.async_copy(table.at[sidx_v.at[pl.ds(k * CHUNK, CHUNK)]],
                         rows, sem)

    def _gwait(k, rows, sem):
        pltpu.make_async_copy(table.at[sidx_v.at[pl.ds(k * CHUNK, CHUNK)]],
                              rows, sem).wait()

    def _dpre(k, didx, ew, sem):
        pltpu.async_copy(dst_hbm.at[pl.ds(ebase + k * CHUNK, CHUNK)],
                         didx, sem)
        pltpu.async_copy(ew_hbm.at[pl.ds(ebase + k * CHUNK, CHUNK)], ew, sem)

    def _dwait(k, didx, ew, sem):
        pltpu.make_async_copy(dst_hbm.at[pl.ds(ebase + k * CHUNK, CHUNK)],
                              didx, sem).wait()
        pltpu.make_async_copy(ew_hbm.at[pl.ds(ebase + k * CHUNK, CHUNK)],
                              ew, sem).wait()

    def _swait(rows, didx, sem):
        pltpu.make_async_copy(rows, acc.at[didx], sem).wait()

    def _mul(ew, rows):
        def _edge16(e16, c):
            wv = ew[pl.ds(e16 * 16, 16)]
            for i in range(16):
                w = jnp.full((16,), wv[i], jnp.float32)
                e = e16 * 16 + i
                for j in range(8):
                    sl = pl.ds(j * 16, 16)
                    rows[e, sl] = rows[e, sl] * w
            return c
        lax.fori_loop(0, CHUNK // 16, _edge16, 0)

    def _proc(k, rows, didx, ew, semg, semi, sems):
        _gwait(k, rows, semg)
        _dwait(k, didx, ew, semi)
        _mul(ew, rows)
        pltpu.async_copy(rows, acc.at[didx], sems, add=True)

    _dpre(0, didx0, ew0, semi0)
    _dpre(1, didx1, ew1, semi1)
    _gather(0, rows0, semg0)
    _gather(1, rows1, semg1)

    def _chunk3(j, carry):
        k = j * 3
        _proc(k, rows0, didx0, ew0, semg0, semi0, sems0)

        @pl.when(k > 0)
        def _():
            _swait(rows2, didx2, sems2)
        _dpre(k + 2, didx2, ew2, semi2)
        _gather(k + 2, rows2, semg2)

        _proc(k + 1, rows1, didx1, ew1, semg1, semi1, sems1)

        _swait(rows0, didx0, sems0)
        _dpre(k + 3, didx0, ew0, semi0)
        _gather(k + 3, rows0, semg0)

        _proc(k + 2, rows2, didx2, ew2, semg2, semi2, sems2)

        _swait(rows1, didx1, sems1)
        _dpre(k + 4, didx1, ew1, semi1)
        _gather(k + 4, rows1, semg1)
        return carry
    lax.fori_loop(0, NCHUNK // 3, _chunk3, 0)

    _proc(NCHUNK - 2, rows0, didx0, ew0, semg0, semi0, sems0)
    _proc(NCHUNK - 1, rows1, didx1, ew1, semg1, semi1, sems1)
    _swait(rows2, didx2, sems2)
    _swait(rows0, didx0, sems0)
    _swait(rows1, didx1, sems1)

    plsc.subcore_barrier()
    pltpu.sync_copy(acc.at[pl.ds(sid * RPT, RPT)],
                    out_hbm.at[pl.ds(cid * NPAD + sid * RPT, RPT)])


_msgpass = functools.partial(
    pl.kernel,
    out_type=jax.ShapeDtypeStruct((2 * NPAD, DH), jnp.float32),
    mesh=plsc.VectorSubcoreMesh(core_axis_name="c", subcore_axis_name="s"),
    scratch_types=[
        pltpu.VMEM((EPT,), jnp.int32),
        pltpu.VMEM((CHUNK,), jnp.int32),
        pltpu.VMEM((CHUNK,), jnp.int32),
        pltpu.VMEM((CHUNK,), jnp.int32),
        pltpu.VMEM((CHUNK,), jnp.float32),
        pltpu.VMEM((CHUNK,), jnp.float32),
        pltpu.VMEM((CHUNK,), jnp.float32),
        pltpu.VMEM((CHUNK, DH), jnp.float32),
        pltpu.VMEM((CHUNK, DH), jnp.float32),
        pltpu.VMEM((CHUNK, DH), jnp.float32),
        pltpu.VMEM_SHARED((NPAD, DH), jnp.float32),
    ] + [pltpu.SemaphoreType.DMA] * 9,
)(_msgpass_body)


def _conv(hl, src, dst, ew):
    table = hl.reshape(2 * N, DH)
    out = _msgpass(table, src, dst, ew)
    return out.reshape(2, NPAD, DH)


def kernel(x, edge_index, edge_weight, batch,
           W0, b0, W1, b1, g0, be0, g1, be1,
           Wp0, bp0, Wp1, bp1, Wp2, bp2):
    src = edge_index[0]
    dst = edge_index[1]
    g0r = g0.reshape(2, 1, DH)
    be0r = be0.reshape(2, 1, DH)
    g1r = g1.reshape(2, 1, DH)
    be1r = be1.reshape(2, 1, DH)
    batch3 = batch.reshape(NB, 1, RB)
    bsum = (bp0 + bp1 + bp2).reshape(1, D_OUT)

    hl0 = _mm0(x, W0)
    conv0 = _conv(hl0, src, dst, ew=edge_weight)
    s0, t0 = _stats(conv0, g0r, be0r)
    hl1 = _mm1(conv0, s0, t0, W1)
    conv1 = _conv(hl1, src, dst, ew=edge_weight)
    s1, t1 = _stats(conv1, g1r, be1r)
    return _pool(x, conv0, s0, t0, conv1, s1, t1, batch3,
                 Wp0, Wp1, Wp2, bsum)
